# baseline (speedup 1.0000x reference)
import functools
import jax
import jax.numpy as jnp
from jax.experimental import pallas as pl

N = 10000
E = 320000
DIN = 128
DREL = 16
NREL = 256
NH = 8
DH = 16

ROWS_BLK = 1000


def _lrelu(x):
    return jnp.where(x >= 0, x, 0.2 * x)


def _dense_body(emb_ref, self_rel_ref, emb_rel_ref, WtT_ref, WhT_ref, WahT_ref,
                WrT_ref, WarT_ref, ba_ref, bg_ref, HSel_ref,
                ptail_ref, thead_ref, lself_ref, vself_ref, ratt_ref, ragg_ref):
    e = emb_ref[...]
    f32 = jnp.float32
    pt = jnp.dot(e, WtT_ref[...], preferred_element_type=f32) + ba_ref[...]
    ph = jnp.dot(e, WhT_ref[...], preferred_element_type=f32)
    ah = jnp.dot(e, WahT_ref[...], preferred_element_type=f32) + bg_ref[...]
    sr = self_rel_ref[...]
    satt = jnp.dot(sr, WrT_ref[...], preferred_element_type=f32)
    sagg = jnp.dot(sr, WarT_ref[...], preferred_element_type=f32)
    ptail_ref[...] = pt
    thead_ref[:, :DIN] = ph
    thead_ref[:, DIN:] = ah
    us = _lrelu(pt + ph + satt)
    lself_ref[...] = jnp.dot(us, HSel_ref[...], preferred_element_type=f32)
    vself_ref[...] = ah + sagg
    er = emb_rel_ref[...]
    ratt_ref[...] = jnp.dot(er, WrT_ref[...], preferred_element_type=f32)
    ragg_ref[...] = jnp.dot(er, WarT_ref[...], preferred_element_type=f32)


def _dense_tables(emb_ent, self_rel, emb_rel, W_attn, b_attn, attn_vec, W_aggr, b_aggr):
    f32 = jnp.float32
    WtT = W_attn[:, :DIN].T
    WhT = W_attn[:, DIN:2 * DIN].T
    WrT = W_attn[:, 2 * DIN:].T
    WahT = W_aggr[:, :DIN].T
    WarT = W_aggr[:, DIN:].T
    av = attn_vec.reshape(NH, DH)
    d_idx = jnp.arange(DIN)
    h_idx = jnp.arange(DH)
    HSel = jnp.where((d_idx[:, None] // DH) == h_idx[None, :],
                     av.reshape(-1)[d_idx][:, None], 0.0).astype(f32)
    grid = (N // ROWS_BLK,)
    full = lambda r, c: pl.BlockSpec((r, c), lambda i: (0, 0))
    blk = lambda c: pl.BlockSpec((ROWS_BLK, c), lambda i: (i, 0))
    return pl.pallas_call(
        _dense_body,
        grid=grid,
        in_specs=[blk(DIN), blk(DREL), full(NREL, DREL), full(DIN, DIN),
                  full(DIN, DIN), full(DIN, DIN), full(DREL, DIN),
                  full(DREL, DIN), full(1, DIN), full(1, DIN), full(DIN, DH)],
        out_specs=[blk(DIN), blk(2 * DIN), blk(DH), blk(DIN),
                   full(NREL, DIN), full(NREL, DIN)],
        out_shape=[
            jax.ShapeDtypeStruct((N, DIN), f32),
            jax.ShapeDtypeStruct((N, 2 * DIN), f32),
            jax.ShapeDtypeStruct((N, DH), f32),
            jax.ShapeDtypeStruct((N, DIN), f32),
            jax.ShapeDtypeStruct((NREL, DIN), f32),
            jax.ShapeDtypeStruct((NREL, DIN), f32),
        ],
    )(emb_ent, self_rel, emb_rel, WtT, WhT, WahT, WrT, WarT,
      b_attn.reshape(1, DIN), b_aggr.reshape(1, DIN), HSel)


def _combine_body(acc_ref, s_ref, vself_ref, rep_ref, out_ref):
    denom = 1.0 + jnp.dot(s_ref[...], rep_ref[...],
                          preferred_element_type=jnp.float32) + 1e-16
    out_ref[...] = (acc_ref[...] + vself_ref[...]) / denom


def _combine(acc, s_pad, V_self):
    rep = (jnp.arange(DIN)[None, :] // DH == jnp.arange(DH)[:, None]).astype(jnp.float32)
    blk = lambda c: pl.BlockSpec((ROWS_BLK, c), lambda i: (i, 0))
    return pl.pallas_call(
        _combine_body,
        grid=(N // ROWS_BLK,),
        in_specs=[blk(DIN), blk(DH), blk(DIN),
                  pl.BlockSpec((DH, DIN), lambda i: (0, 0))],
        out_specs=blk(DIN),
        out_shape=jax.ShapeDtypeStruct((N, DIN), jnp.float32),
    )(acc, s_pad, V_self, rep)


def kernel(emb_ent, edge_index, edge_type, emb_rel, W_attn, b_attn, attn_vec, W_aggr, b_aggr):
    head = edge_index[0]
    tail = edge_index[1]

    deg = jnp.zeros((N,), jnp.float32).at[tail].add(1.0)
    rel_sum = jnp.zeros((N, DREL), jnp.float32).at[tail].add(emb_rel[edge_type])
    self_rel = rel_sum / deg[:, None]

    P_tail, T_head, l_self, V_self, R_att, R_agg = _dense_tables(
        emb_ent, self_rel, emb_rel, W_attn, b_attn, attn_vec, W_aggr, b_aggr)

    av = attn_vec.reshape(NH, DH)
    u = _lrelu(P_tail[tail] + T_head[head, :DIN] + R_att[edge_type])
    l = (u.reshape(-1, NH, DH) * av).sum(-1)
    w = jnp.exp(l - l_self[tail, :NH])
    v = T_head[head, DIN:] + R_agg[edge_type]
    scaled = (w[:, :, None] * v.reshape(-1, NH, DH)).reshape(-1, DIN)
    acc = jnp.zeros((N, DIN), jnp.float32).at[tail].add(scaled)
    s = jnp.zeros((N, DH), jnp.float32).at[tail, :NH].add(w)

    return _combine(acc, s, V_self)


# speedup vs baseline: 102.4411x; 102.4411x over previous
import functools
import jax
import jax.numpy as jnp
from jax import lax
from jax.experimental import pallas as pl
from jax.experimental.pallas import tpu as pltpu
from jax.experimental.pallas import tpu_sc as plsc

N = 10000
E = 320000
DIN = 128
DREL = 16
NREL = 256
NH = 8
DH = 16

NC = 2
NS = 16
NW = NC * NS
EPW = E // NW
N_PAD = 10240
NPS = N_PAD // NS
NQ = N_PAD // 4
NQS = NQ // NS
NG = N_PAD // 16
NGS = NG // NS

ROWS_BLK = 1000


def _lrelu(x):
    return jnp.where(x >= 0, x, 0.2 * x)


_BA = 400


def _selfrel_sc(edge_ix_flat, edge_type, aug4):
    mesh = plsc.VectorSubcoreMesh(core_axis_name="c", subcore_axis_name="s")

    @functools.partial(
        pl.kernel,
        out_type=jax.ShapeDtypeStruct((NC, NQ, DIN), jnp.float32),
        mesh=mesh,
        scratch_types=[
            pltpu.VMEM((_BA,), jnp.int32),
            pltpu.VMEM((_BA,), jnp.int32),
            pltpu.VMEM((_BA,), jnp.int32),
            pltpu.VMEM((_BA, DIN), jnp.float32),
            pltpu.VMEM_SHARED((NQ, DIN), jnp.float32),
            pltpu.SemaphoreType.DMA,
        ],
        compiler_params=pltpu.CompilerParams(needs_layout_passes=False),
    )
    def body(ei_hbm, et_hbm, aug_hbm, out_hbm, tails_v, gidx_v, prow_v,
             rows_v, acc_sh, sem):
        cid = lax.axis_index("c")
        sid = lax.axis_index("s")
        wid = sid * NC + cid
        zvec = jnp.zeros((16,), jnp.float32)

        def zstore(j, _):
            rows_v[j // 8, pl.ds((j % 8) * 16, 16)] = zvec
            return 0
        lax.fori_loop(0, NQS * 8, zstore, 0)
        pltpu.sync_copy(rows_v.at[pl.ds(0, NQS)],
                        acc_sh.at[pl.ds(sid * NQS, NQS)])
        plsc.subcore_barrier()

        def chunk(k, _):
            base = wid * EPW + k * _BA
            pltpu.sync_copy(ei_hbm.at[pl.ds(E + base, _BA)], tails_v)
            pltpu.sync_copy(et_hbm.at[pl.ds(base, _BA)], gidx_v)

            def mix(j, _):
                t = tails_v[pl.ds(j * 16, 16)]
                ty = gidx_v[pl.ds(j * 16, 16)]
                gidx_v[pl.ds(j * 16, 16)] = ty * 4 + (t & 3)
                prow_v[pl.ds(j * 16, 16)] = lax.shift_right_logical(t, 2)
                return 0
            lax.fori_loop(0, _BA // 16, mix, 0)

            pltpu.async_copy(aug_hbm.at[gidx_v], rows_v, sem).wait()
            pltpu.sync_copy(rows_v, acc_sh.at[prow_v], add=True)
            return 0

        lax.fori_loop(0, EPW // _BA, chunk, 0)
        plsc.subcore_barrier()
        pltpu.sync_copy(acc_sh.at[pl.ds(sid * NQS, NQS)],
                        out_hbm.at[cid, pl.ds(sid * NQS, NQS)])

    return body(edge_ix_flat, edge_type, aug4)


_BC = 80


def _edge_sc(edge_ix_flat, edge_type, p_tail, p_head, a_head, r_att, r_agg,
             av_flat):
    mesh = plsc.VectorSubcoreMesh(core_axis_name="c", subcore_axis_name="s")

    @functools.partial(
        pl.kernel,
        out_type=(jax.ShapeDtypeStruct((NC, N_PAD, DIN), jnp.float32),
                  jax.ShapeDtypeStruct((NC, NG, DIN), jnp.float32)),
        mesh=mesh,
        scratch_types=[
            pltpu.VMEM((_BC,), jnp.int32),
            pltpu.VMEM((_BC,), jnp.int32),
            pltpu.VMEM((_BC,), jnp.int32),
            pltpu.VMEM((_BC,), jnp.int32),
            pltpu.VMEM((_BC, DIN), jnp.float32),
            pltpu.VMEM((_BC, DIN), jnp.float32),
            pltpu.VMEM((_BC, DIN), jnp.float32),
            pltpu.VMEM((_BC, DIN), jnp.float32),
            pltpu.VMEM((DIN,), jnp.float32),
            pltpu.VMEM((_BC * NH,), jnp.float32),
            pltpu.VMEM((DIN,), jnp.float32),
            pltpu.VMEM_SHARED((N_PAD, DIN), jnp.float32),
            pltpu.VMEM_SHARED((NG, DIN), jnp.float32),
            pltpu.SemaphoreType.DMA,
            pltpu.SemaphoreType.DMA,
            pltpu.SemaphoreType.DMA,
        ],
        compiler_params=pltpu.CompilerParams(needs_layout_passes=False),
    )
    def body(ei_hbm, et_hbm, ptail_hbm, phead_hbm, ahead_hbm, ratt_hbm,
             ragg_hbm, av_hbm, acc_out, s_out, tails_v, heads_v, types_v,
             tgrp_v, bufA, bufB, bufC, stage_s, logit_st, w_ch, av_v,
             acc_sh, s_sh, sem1, sem2, sem3):
        cid = lax.axis_index("c")
        sid = lax.axis_index("s")
        wid = sid * NC + cid
        zvec = jnp.zeros((16,), jnp.float32)
        iota = lax.iota(jnp.int32, 16)
        lane15 = iota == 15

        pltpu.sync_copy(av_hbm, av_v)

        def zstore(j, _):
            bufA[j // 8, pl.ds((j % 8) * 16, 16)] = zvec
            return 0
        lax.fori_loop(0, _BC * 8, zstore, 0)
        for q in range(NPS // _BC):
            pltpu.sync_copy(bufA, acc_sh.at[pl.ds(sid * NPS + q * _BC, _BC)])
        pltpu.sync_copy(bufA.at[pl.ds(0, NGS)], s_sh.at[pl.ds(sid * NGS, NGS)])
        plsc.subcore_barrier()

        def chunk(k, _):
            base = wid * EPW + k * _BC
            pltpu.sync_copy(ei_hbm.at[pl.ds(base, _BC)], heads_v)
            pltpu.sync_copy(ei_hbm.at[pl.ds(E + base, _BC)], tails_v)
            pltpu.sync_copy(et_hbm.at[pl.ds(base, _BC)], types_v)
            c1 = pltpu.async_copy(ptail_hbm.at[tails_v], bufA, sem1)
            c2 = pltpu.async_copy(phead_hbm.at[heads_v], bufB, sem2)
            c3 = pltpu.async_copy(ratt_hbm.at[types_v], bufC, sem3)
            c1.wait()
            c2.wait()
            c3.wait()

            def group(g, _):
                def p1(el, _):
                    e = g * 16 + el
                    for h in range(NH):
                        stage_s[e, pl.ds(h * 16, 16)] = zvec
                        u = (bufA[e, pl.ds(h * 16, 16)]
                             + bufB[e, pl.ds(h * 16, 16)]
                             + bufC[e, pl.ds(h * 16, 16)])
                        u = jnp.where(u >= 0, u, 0.2 * u)
                        t = u * av_v[pl.ds(h * 16, 16)]
                        cs = plsc.cumsum(t)
                        plsc.store_scatter(
                            logit_st,
                            [jnp.full((16,), h * 16 + el, jnp.int32)],
                            cs, mask=lane15)
                    return 0
                lax.fori_loop(0, 16, p1, 0)

                tl = tails_v[pl.ds(g * 16, 16)]
                tgrp_v[pl.ds(g * 16, 16)] = lax.shift_right_logical(tl, 4)
                scol = (tl & 15) * 8
                erow = g * 16 + iota
                for h in range(NH):
                    wv = jnp.exp(logit_st[pl.ds(h * 16, 16)])
                    w_ch[pl.ds((g * NH + h) * 16, 16)] = wv
                    plsc.store_scatter(stage_s, [erow, scol + h], wv)
                return 0

            lax.fori_loop(0, _BC // 16, group, 0)

            c4 = pltpu.async_copy(ahead_hbm.at[heads_v], bufB, sem2)
            c5 = pltpu.async_copy(ragg_hbm.at[types_v], bufC, sem3)
            c4.wait()
            c5.wait()

            def p3(e, _):
                g = lax.shift_right_logical(e, 4)
                el = e & 15
                for h in range(NH):
                    wsp = plsc.load_gather(
                        w_ch, [jnp.full((16,), (g * NH + h) * 16 + el,
                                        jnp.int32)])
                    v = bufB[e, pl.ds(h * 16, 16)] + bufC[e, pl.ds(h * 16, 16)]
                    bufA[e, pl.ds(h * 16, 16)] = v * wsp
                return 0
            lax.fori_loop(0, _BC, p3, 0)

            pltpu.sync_copy(bufA, acc_sh.at[tails_v], add=True)
            pltpu.sync_copy(stage_s, s_sh.at[tgrp_v], add=True)
            return 0

        lax.fori_loop(0, EPW // _BC, chunk, 0)
        plsc.subcore_barrier()
        pltpu.sync_copy(acc_sh.at[pl.ds(sid * NPS, NPS)],
                        acc_out.at[cid, pl.ds(sid * NPS, NPS)])
        pltpu.sync_copy(s_sh.at[pl.ds(sid * NGS, NGS)],
                        s_out.at[cid, pl.ds(sid * NGS, NGS)])

    return body(edge_ix_flat, edge_type, p_tail, p_head, a_head, r_att,
                r_agg, av_flat)


def _dense_body(emb_ref, a0_ref, a1_ref, emb_rel_ref, WtT_ref, WhT_ref, WahT_ref,
                WrT_ref, WarT_ref, ba_ref, bg_ref, HSel_ref,
                ptail_ref, phead_ref, ahead_ref, lself_ref, vself_ref,
                ratt_ref, ragg_ref):
    e = emb_ref[...]
    f32 = jnp.float32
    pt = jnp.dot(e, WtT_ref[...], preferred_element_type=f32) + ba_ref[...]
    ph = jnp.dot(e, WhT_ref[...], preferred_element_type=f32)
    ah = jnp.dot(e, WahT_ref[...], preferred_element_type=f32) + bg_ref[...]
    asum = a0_ref[...] + a1_ref[...]
    sr = asum[:, :DREL] / asum[:, DREL:DREL + 1]
    satt = jnp.dot(sr, WrT_ref[...], preferred_element_type=f32)
    sagg = jnp.dot(sr, WarT_ref[...], preferred_element_type=f32)
    ptail_ref[...] = pt
    phead_ref[...] = ph
    ahead_ref[...] = ah
    us = _lrelu(pt + ph + satt)
    lself_ref[...] = jnp.dot(us, HSel_ref[...], preferred_element_type=f32)
    vself_ref[...] = ah + sagg
    er = emb_rel_ref[...]
    ratt_ref[...] = jnp.dot(er, WrT_ref[...], preferred_element_type=f32)
    ragg_ref[...] = jnp.dot(er, WarT_ref[...], preferred_element_type=f32)


def _dense_tables(emb_ent, a0, a1, emb_rel, W_attn, b_attn, attn_vec, W_aggr, b_aggr):
    f32 = jnp.float32
    WtT = W_attn[:, :DIN].T
    WhT = W_attn[:, DIN:2 * DIN].T
    WrT = W_attn[:, 2 * DIN:].T
    WahT = W_aggr[:, :DIN].T
    WarT = W_aggr[:, DIN:].T
    av = attn_vec.reshape(NH, DH)
    d_idx = jnp.arange(DIN)
    h_idx = jnp.arange(DH)
    HSel = jnp.where((d_idx[:, None] // DH) == h_idx[None, :],
                     av.reshape(-1)[d_idx][:, None], 0.0).astype(f32)
    grid = (N // ROWS_BLK,)
    full = lambda r, c: pl.BlockSpec((r, c), lambda i: (0, 0))
    blk = lambda c: pl.BlockSpec((ROWS_BLK, c), lambda i: (i, 0))
    return pl.pallas_call(
        _dense_body,
        grid=grid,
        in_specs=[blk(DIN), blk(2 * DREL), blk(2 * DREL), full(NREL, DREL),
                  full(DIN, DIN), full(DIN, DIN), full(DIN, DIN), full(DREL, DIN),
                  full(DREL, DIN), full(1, DIN), full(1, DIN), full(DIN, DH)],
        out_specs=[blk(DIN), blk(DIN), blk(DIN), blk(DH), blk(DIN),
                   full(NREL, DIN), full(NREL, DIN)],
        out_shape=[
            jax.ShapeDtypeStruct((N, DIN), f32),
            jax.ShapeDtypeStruct((N, DIN), f32),
            jax.ShapeDtypeStruct((N, DIN), f32),
            jax.ShapeDtypeStruct((N, DH), f32),
            jax.ShapeDtypeStruct((N, DIN), f32),
            jax.ShapeDtypeStruct((NREL, DIN), f32),
            jax.ShapeDtypeStruct((NREL, DIN), f32),
        ],
    )(emb_ent, a0, a1, emb_rel, WtT, WhT, WahT, WrT, WarT,
      b_attn.reshape(1, DIN), b_aggr.reshape(1, DIN), HSel)


def _combine_body(a0_ref, a1_ref, s_ref, lself_ref, vself_ref, rep_ref, out_ref):
    f32 = jnp.float32
    sw = jnp.dot(s_ref[...], rep_ref[...], preferred_element_type=f32)
    lw = jnp.dot(lself_ref[...], rep_ref[...], preferred_element_type=f32)
    z = jnp.exp(-lw)
    acc = a0_ref[...] + a1_ref[...]
    out_ref[...] = (acc * z + vself_ref[...]) / (1.0 + sw * z + 1e-16)


def _combine(acc0, acc1, s16, l_self, V_self):
    rep = (jnp.arange(DIN)[None, :] // DH == jnp.arange(DH)[:, None]).astype(jnp.float32)
    blk = lambda c: pl.BlockSpec((ROWS_BLK, c), lambda i: (i, 0))
    return pl.pallas_call(
        _combine_body,
        grid=(N // ROWS_BLK,),
        in_specs=[blk(DIN), blk(DIN), blk(DH), blk(DH), blk(DIN),
                  pl.BlockSpec((DH, DIN), lambda i: (0, 0))],
        out_specs=blk(DIN),
        out_shape=jax.ShapeDtypeStruct((N, DIN), jnp.float32),
    )(acc0, acc1, s16, l_self, V_self, rep)


def kernel(emb_ent, edge_index, edge_type, emb_rel, W_attn, b_attn, attn_vec, W_aggr, b_aggr):
    f32 = jnp.float32
    ei_flat = edge_index.reshape(-1)

    base_rel = jnp.concatenate(
        [emb_rel, jnp.ones((NREL, 1), f32), jnp.zeros((NREL, 15), f32)],
        axis=1)
    aug4 = jnp.einsum('qr,tc->tqrc', jnp.eye(4, dtype=f32),
                      base_rel).reshape(NREL * 4, DIN)
    acc_sr = _selfrel_sc(ei_flat, edge_type, aug4)
    acc_sr = acc_sr.reshape(NC, N_PAD, 2 * DREL)[:, :N]

    P_tail, P_head, A_head, l_self, V_self, R_att, R_agg = _dense_tables(
        emb_ent, acc_sr[0], acc_sr[1], emb_rel, W_attn, b_attn, attn_vec,
        W_aggr, b_aggr)

    acc_out, s_out = _edge_sc(ei_flat, edge_type, P_tail, P_head, A_head,
                              R_att, R_agg, attn_vec.reshape(-1))
    acc0 = acc_out[0, :N]
    acc1 = acc_out[1, :N]
    s8 = (s_out[0] + s_out[1]).reshape(N_PAD, NH)[:N]
    s16 = jnp.concatenate([s8, jnp.zeros((N, NH), f32)], axis=1)

    return _combine(acc0, acc1, s16, l_self, V_self)


# speedup vs baseline: 153.6900x; 1.5003x over previous
import functools
import jax
import jax.numpy as jnp
from jax import lax
from jax.experimental import pallas as pl
from jax.experimental.pallas import tpu as pltpu
from jax.experimental.pallas import tpu_sc as plsc

N = 10000
E = 320000
DIN = 128
DREL = 16
NREL = 256
NH = 8
DH = 16

NC = 2
NS = 16
NW = NC * NS
EPW = E // NW
N_PAD = 10240
NPS = N_PAD // NS
NQ = N_PAD // 4
NQS = NQ // NS
NG = N_PAD // 16
NGS = NG // NS

ROWS_BLK = 1000


def _lrelu(x):
    return jnp.where(x >= 0, x, 0.2 * x)


_BA = 400


def _selfrel_sc(edge_ix_flat, edge_type, aug4):
    mesh = plsc.VectorSubcoreMesh(core_axis_name="c", subcore_axis_name="s")

    @functools.partial(
        pl.kernel,
        out_type=jax.ShapeDtypeStruct((NC, NQ, DIN), jnp.float32),
        mesh=mesh,
        scratch_types=[
            pltpu.VMEM((_BA,), jnp.int32),
            pltpu.VMEM((_BA,), jnp.int32),
            pltpu.VMEM((_BA,), jnp.int32),
            pltpu.VMEM((_BA, DIN), jnp.float32),
            pltpu.VMEM_SHARED((NQ, DIN), jnp.float32),
            pltpu.SemaphoreType.DMA,
        ],
        compiler_params=pltpu.CompilerParams(needs_layout_passes=False),
    )
    def body(ei_hbm, et_hbm, aug_hbm, out_hbm, tails_v, gidx_v, prow_v,
             rows_v, acc_sh, sem):
        cid = lax.axis_index("c")
        sid = lax.axis_index("s")
        wid = sid * NC + cid
        zvec = jnp.zeros((16,), jnp.float32)

        def zstore(j, _):
            rows_v[j // 8, pl.ds((j % 8) * 16, 16)] = zvec
            return 0
        lax.fori_loop(0, NQS * 8, zstore, 0)
        pltpu.sync_copy(rows_v.at[pl.ds(0, NQS)],
                        acc_sh.at[pl.ds(sid * NQS, NQS)])
        plsc.subcore_barrier()

        def chunk(k, _):
            base = wid * EPW + k * _BA
            pltpu.sync_copy(ei_hbm.at[pl.ds(E + base, _BA)], tails_v)
            pltpu.sync_copy(et_hbm.at[pl.ds(base, _BA)], gidx_v)

            def mix(j, _):
                t = tails_v[pl.ds(j * 16, 16)]
                ty = gidx_v[pl.ds(j * 16, 16)]
                gidx_v[pl.ds(j * 16, 16)] = ty * 4 + (t & 3)
                prow_v[pl.ds(j * 16, 16)] = lax.shift_right_logical(t, 2)
                return 0
            lax.fori_loop(0, _BA // 16, mix, 0)

            pltpu.async_copy(aug_hbm.at[gidx_v], rows_v, sem).wait()
            pltpu.sync_copy(rows_v, acc_sh.at[prow_v], add=True)
            return 0

        lax.fori_loop(0, EPW // _BA, chunk, 0)
        plsc.subcore_barrier()
        pltpu.sync_copy(acc_sh.at[pl.ds(sid * NQS, NQS)],
                        out_hbm.at[cid, pl.ds(sid * NQS, NQS)])

    return body(edge_ix_flat, edge_type, aug4)


_BC = 32
_NCH = E // _BC
_TRIPS = (_NCH + NW - 1) // NW + 1


def _edge_sc(edge_ix_flat, edge_type, p_tail, p_head, a_head, r_att, r_agg,
             av_flat):
    mesh = plsc.VectorSubcoreMesh(core_axis_name="c", subcore_axis_name="s")

    @functools.partial(
        pl.kernel,
        out_type=(jax.ShapeDtypeStruct((NC, N_PAD, DIN), jnp.float32),
                  jax.ShapeDtypeStruct((NC, NG, DIN), jnp.float32)),
        mesh=mesh,
        scratch_types=[
            pltpu.VMEM((2, _BC), jnp.int32),
            pltpu.VMEM((2, _BC), jnp.int32),
            pltpu.VMEM((2, _BC), jnp.int32),
            pltpu.VMEM((_BC,), jnp.int32),
            pltpu.VMEM((2, _BC, DIN), jnp.float32),
            pltpu.VMEM((2, _BC, DIN), jnp.float32),
            pltpu.VMEM((2, _BC, DIN), jnp.float32),
            pltpu.VMEM((_BC, DIN), jnp.float32),
            pltpu.VMEM((DIN,), jnp.float32),
            pltpu.VMEM((_BC * NH,), jnp.float32),
            pltpu.VMEM((DIN,), jnp.float32),
            pltpu.VMEM_SHARED((N_PAD, DIN), jnp.float32),
            pltpu.VMEM_SHARED((NG, DIN), jnp.float32),
            pltpu.SemaphoreType.DMA,
            pltpu.SemaphoreType.DMA,
            pltpu.SemaphoreType.DMA,
            pltpu.SemaphoreType.DMA,
            pltpu.SemaphoreType.DMA,
            pltpu.SemaphoreType.DMA,
            pltpu.SemaphoreType.DMA,
            pltpu.SemaphoreType.DMA,
        ],
        compiler_params=pltpu.CompilerParams(needs_layout_passes=False),
    )
    def body(ei_hbm, et_hbm, ptail_hbm, phead_hbm, ahead_hbm, ratt_hbm,
             ragg_hbm, av_hbm, acc_out, s_out, tails_v, heads_v, types_v,
             tgrp_v, bufA, bufB, bufC, stage_s, logit_st, w_ch, av_v,
             acc_sh, s_sh, semi0, semi1, semA0, semA1, semB0, semB1,
             semC0, semC1):
        cid = lax.axis_index("c")
        sid = lax.axis_index("s")
        wid = sid * NC + cid
        zvec = jnp.zeros((16,), jnp.float32)
        iota = lax.iota(jnp.int32, 16)
        lane15 = iota == 15
        semi = (semi0, semi1)
        semA = (semA0, semA1)
        semB = (semB0, semB1)
        semC = (semC0, semC1)

        pltpu.sync_copy(av_hbm, av_v)

        def zstore(j, _):
            bufA[0, j // 8, pl.ds((j % 8) * 16, 16)] = zvec
            return 0
        lax.fori_loop(0, _BC * 8, zstore, 0)
        for q in range(NPS // _BC):
            pltpu.sync_copy(bufA.at[0],
                            acc_sh.at[pl.ds(sid * NPS + q * _BC, _BC)])
        pltpu.sync_copy(bufA.at[0], s_sh.at[pl.ds(sid * NGS, _BC)])
        pltpu.sync_copy(bufA.at[0, pl.ds(0, NGS - _BC)],
                        s_sh.at[pl.ds(sid * NGS + _BC, NGS - _BC)])
        plsc.subcore_barrier()

        def idx_issue(c, p):
            base = c * _BC
            pltpu.async_copy(ei_hbm.at[pl.ds(base, _BC)], heads_v.at[p],
                             semi[p])
            pltpu.async_copy(ei_hbm.at[pl.ds(E + base, _BC)], tails_v.at[p],
                             semi[p])
            pltpu.async_copy(et_hbm.at[pl.ds(base, _BC)], types_v.at[p],
                             semi[p])

        def idx_wait(p):
            for dst in (heads_v, tails_v, types_v):
                pltpu.make_async_copy(et_hbm.at[pl.ds(0, _BC)], dst.at[p],
                                      semi[p]).wait()

        def gather_issue(p):
            pltpu.async_copy(ptail_hbm.at[tails_v.at[p]], bufA.at[p], semA[p])
            pltpu.async_copy(phead_hbm.at[heads_v.at[p]], bufB.at[p], semB[p])
            pltpu.async_copy(ratt_hbm.at[types_v.at[p]], bufC.at[p], semC[p])

        def gather_wait(p):
            pltpu.make_async_copy(ptail_hbm.at[tails_v.at[p]], bufA.at[p],
                                  semA[p]).wait()
            pltpu.make_async_copy(phead_hbm.at[heads_v.at[p]], bufB.at[p],
                                  semB[p]).wait()
            pltpu.make_async_copy(ratt_hbm.at[types_v.at[p]], bufC.at[p],
                                  semC[p]).wait()

        def compute(p):
            for g in range(_BC // 16):
                @plsc.parallel_loop(0, 16, unroll=2)
                def p1(el):
                    e = g * 16 + el
                    for h in range(NH):
                        stage_s[e, pl.ds(h * 16, 16)] = zvec
                        u = (bufA[p, e, pl.ds(h * 16, 16)]
                             + bufB[p, e, pl.ds(h * 16, 16)]
                             + bufC[p, e, pl.ds(h * 16, 16)])
                        u = jnp.where(u >= 0, u, 0.2 * u)
                        t = u * av_v[pl.ds(h * 16, 16)]
                        cs = plsc.cumsum(t)
                        plsc.store_scatter(
                            logit_st,
                            [jnp.full((16,), h * 16 + el, jnp.int32)],
                            cs, mask=lane15)

                tl = tails_v[p, pl.ds(g * 16, 16)]
                tgrp_v[pl.ds(g * 16, 16)] = lax.shift_right_logical(tl, 4)
                scol = (tl & 15) * 8
                erow = g * 16 + iota
                for h in range(NH):
                    wv = jnp.exp(logit_st[pl.ds(h * 16, 16)])
                    w_ch[pl.ds((g * NH + h) * 16, 16)] = wv
                    plsc.store_scatter(stage_s, [erow, scol + h], wv)

            c4 = pltpu.async_copy(ahead_hbm.at[heads_v.at[p]], bufB.at[p],
                                  semB[p])
            c5 = pltpu.async_copy(ragg_hbm.at[types_v.at[p]], bufC.at[p],
                                  semC[p])
            c4.wait()
            c5.wait()

            @plsc.parallel_loop(0, _BC, unroll=2)
            def p3(e):
                g = lax.shift_right_logical(e, 4)
                el = e & 15
                for h in range(NH):
                    wsp = plsc.load_gather(
                        w_ch, [jnp.full((16,), (g * NH + h) * 16 + el,
                                        jnp.int32)])
                    v = (bufB[p, e, pl.ds(h * 16, 16)]
                         + bufC[p, e, pl.ds(h * 16, 16)])
                    bufA[p, e, pl.ds(h * 16, 16)] = v * wsp

            pltpu.sync_copy(bufA.at[p], acc_sh.at[tails_v.at[p]], add=True)
            pltpu.sync_copy(stage_s, s_sh.at[tgrp_v], add=True)

        idx_issue(wid, 0)
        idx_wait(0)
        gather_issue(0)

        def outer(j2, _):
            for par in range(2):
                j = j2 * 2 + par
                c = wid + j * NW
                cn = wid + (j + 1) * NW

                @pl.when(cn < _NCH)
                def _():
                    idx_issue(cn, par ^ 1)

                @pl.when(c < _NCH)
                def _():
                    gather_wait(par)
                    compute(par)

                @pl.when(cn < _NCH)
                def _():
                    idx_wait(par ^ 1)
                    gather_issue(par ^ 1)
            return 0

        lax.fori_loop(0, (_TRIPS + 1) // 2, outer, 0)
        plsc.subcore_barrier()
        pltpu.sync_copy(acc_sh.at[pl.ds(sid * NPS, NPS)],
                        acc_out.at[cid, pl.ds(sid * NPS, NPS)])
        pltpu.sync_copy(s_sh.at[pl.ds(sid * NGS, NGS)],
                        s_out.at[cid, pl.ds(sid * NGS, NGS)])

    return body(edge_ix_flat, edge_type, p_tail, p_head, a_head, r_att,
                r_agg, av_flat)


def _dense_body(emb_ref, a0_ref, a1_ref, emb_rel_ref, WtT_ref, WhT_ref, WahT_ref,
                WrT_ref, WarT_ref, ba_ref, bg_ref, HSel_ref,
                ptail_ref, phead_ref, ahead_ref, lself_ref, vself_ref,
                ratt_ref, ragg_ref):
    e = emb_ref[...]
    f32 = jnp.float32
    pt = jnp.dot(e, WtT_ref[...], preferred_element_type=f32) + ba_ref[...]
    ph = jnp.dot(e, WhT_ref[...], preferred_element_type=f32)
    ah = jnp.dot(e, WahT_ref[...], preferred_element_type=f32) + bg_ref[...]
    asum = a0_ref[...] + a1_ref[...]
    sr = asum[:, :DREL] / asum[:, DREL:DREL + 1]
    satt = jnp.dot(sr, WrT_ref[...], preferred_element_type=f32)
    sagg = jnp.dot(sr, WarT_ref[...], preferred_element_type=f32)
    ptail_ref[...] = pt
    phead_ref[...] = ph
    ahead_ref[...] = ah
    us = _lrelu(pt + ph + satt)
    lself_ref[...] = jnp.dot(us, HSel_ref[...], preferred_element_type=f32)
    vself_ref[...] = ah + sagg
    er = emb_rel_ref[...]
    ratt_ref[...] = jnp.dot(er, WrT_ref[...], preferred_element_type=f32)
    ragg_ref[...] = jnp.dot(er, WarT_ref[...], preferred_element_type=f32)


def _dense_tables(emb_ent, a0, a1, emb_rel, W_attn, b_attn, attn_vec, W_aggr, b_aggr):
    f32 = jnp.float32
    WtT = W_attn[:, :DIN].T
    WhT = W_attn[:, DIN:2 * DIN].T
    WrT = W_attn[:, 2 * DIN:].T
    WahT = W_aggr[:, :DIN].T
    WarT = W_aggr[:, DIN:].T
    av = attn_vec.reshape(NH, DH)
    d_idx = jnp.arange(DIN)
    h_idx = jnp.arange(DH)
    HSel = jnp.where((d_idx[:, None] // DH) == h_idx[None, :],
                     av.reshape(-1)[d_idx][:, None], 0.0).astype(f32)
    grid = (N // ROWS_BLK,)
    full = lambda r, c: pl.BlockSpec((r, c), lambda i: (0, 0))
    blk = lambda c: pl.BlockSpec((ROWS_BLK, c), lambda i: (i, 0))
    return pl.pallas_call(
        _dense_body,
        grid=grid,
        in_specs=[blk(DIN), blk(2 * DREL), blk(2 * DREL), full(NREL, DREL),
                  full(DIN, DIN), full(DIN, DIN), full(DIN, DIN), full(DREL, DIN),
                  full(DREL, DIN), full(1, DIN), full(1, DIN), full(DIN, DH)],
        out_specs=[blk(DIN), blk(DIN), blk(DIN), blk(DH), blk(DIN),
                   full(NREL, DIN), full(NREL, DIN)],
        out_shape=[
            jax.ShapeDtypeStruct((N, DIN), f32),
            jax.ShapeDtypeStruct((N, DIN), f32),
            jax.ShapeDtypeStruct((N, DIN), f32),
            jax.ShapeDtypeStruct((N, DH), f32),
            jax.ShapeDtypeStruct((N, DIN), f32),
            jax.ShapeDtypeStruct((NREL, DIN), f32),
            jax.ShapeDtypeStruct((NREL, DIN), f32),
        ],
    )(emb_ent, a0, a1, emb_rel, WtT, WhT, WahT, WrT, WarT,
      b_attn.reshape(1, DIN), b_aggr.reshape(1, DIN), HSel)


def _combine_body(a0_ref, a1_ref, s_ref, lself_ref, vself_ref, rep_ref, out_ref):
    f32 = jnp.float32
    sw = jnp.dot(s_ref[...], rep_ref[...], preferred_element_type=f32)
    lw = jnp.dot(lself_ref[...], rep_ref[...], preferred_element_type=f32)
    z = jnp.exp(-lw)
    acc = a0_ref[...] + a1_ref[...]
    out_ref[...] = (acc * z + vself_ref[...]) / (1.0 + sw * z + 1e-16)


def _combine(acc0, acc1, s16, l_self, V_self):
    rep = (jnp.arange(DIN)[None, :] // DH == jnp.arange(DH)[:, None]).astype(jnp.float32)
    blk = lambda c: pl.BlockSpec((ROWS_BLK, c), lambda i: (i, 0))
    return pl.pallas_call(
        _combine_body,
        grid=(N // ROWS_BLK,),
        in_specs=[blk(DIN), blk(DIN), blk(DH), blk(DH), blk(DIN),
                  pl.BlockSpec((DH, DIN), lambda i: (0, 0))],
        out_specs=blk(DIN),
        out_shape=jax.ShapeDtypeStruct((N, DIN), jnp.float32),
    )(acc0, acc1, s16, l_self, V_self, rep)


def kernel(emb_ent, edge_index, edge_type, emb_rel, W_attn, b_attn, attn_vec, W_aggr, b_aggr):
    f32 = jnp.float32
    ei_flat = edge_index.reshape(-1)

    base_rel = jnp.concatenate(
        [emb_rel, jnp.ones((NREL, 1), f32), jnp.zeros((NREL, 15), f32)],
        axis=1)
    aug4 = jnp.einsum('qr,tc->tqrc', jnp.eye(4, dtype=f32),
                      base_rel).reshape(NREL * 4, DIN)
    acc_sr = _selfrel_sc(ei_flat, edge_type, aug4)
    acc_sr = acc_sr.reshape(NC, N_PAD, 2 * DREL)[:, :N]

    P_tail, P_head, A_head, l_self, V_self, R_att, R_agg = _dense_tables(
        emb_ent, acc_sr[0], acc_sr[1], emb_rel, W_attn, b_attn, attn_vec,
        W_aggr, b_aggr)

    acc_out, s_out = _edge_sc(ei_flat, edge_type, P_tail, P_head, A_head,
                              R_att, R_agg, attn_vec.reshape(-1))
    acc0 = acc_out[0, :N]
    acc1 = acc_out[1, :N]
    s8 = (s_out[0] + s_out[1]).reshape(N_PAD, NH)[:N]
    s16 = jnp.concatenate([s8, jnp.zeros((N, NH), f32)], axis=1)

    return _combine(acc0, acc1, s16, l_self, V_self)


# speedup vs baseline: 161.1093x; 1.0483x over previous
import functools
import jax
import jax.numpy as jnp
from jax import lax
from jax.experimental import pallas as pl
from jax.experimental.pallas import tpu as pltpu
from jax.experimental.pallas import tpu_sc as plsc

N = 10000
E = 320000
DIN = 128
DREL = 16
NREL = 256
NH = 8
DH = 16

NC = 2
NS = 16
NW = NC * NS
EPW = E // NW
N_PAD = 10240
NPS = N_PAD // NS
NQ = N_PAD // 4
NQS = NQ // NS
NG = N_PAD // 16
NGS = NG // NS

ROWS_BLK = 1000


def _lrelu(x):
    return jnp.where(x >= 0, x, 0.2 * x)


_BA = 400


def _selfrel_sc(edge_ix_flat, edge_type, aug4):
    mesh = plsc.VectorSubcoreMesh(core_axis_name="c", subcore_axis_name="s")

    @functools.partial(
        pl.kernel,
        out_type=jax.ShapeDtypeStruct((NC, NQ, DIN), jnp.float32),
        mesh=mesh,
        scratch_types=[
            pltpu.VMEM((_BA,), jnp.int32),
            pltpu.VMEM((_BA,), jnp.int32),
            pltpu.VMEM((_BA,), jnp.int32),
            pltpu.VMEM((_BA, DIN), jnp.float32),
            pltpu.VMEM_SHARED((NQ, DIN), jnp.float32),
            pltpu.SemaphoreType.DMA,
        ],
        compiler_params=pltpu.CompilerParams(needs_layout_passes=False),
    )
    def body(ei_hbm, et_hbm, aug_hbm, out_hbm, tails_v, gidx_v, prow_v,
             rows_v, acc_sh, sem):
        cid = lax.axis_index("c")
        sid = lax.axis_index("s")
        wid = sid * NC + cid
        zvec = jnp.zeros((16,), jnp.float32)

        def zstore(j, _):
            rows_v[j // 8, pl.ds((j % 8) * 16, 16)] = zvec
            return 0
        lax.fori_loop(0, NQS * 8, zstore, 0)
        pltpu.sync_copy(rows_v.at[pl.ds(0, NQS)],
                        acc_sh.at[pl.ds(sid * NQS, NQS)])
        plsc.subcore_barrier()

        def chunk(k, _):
            base = wid * EPW + k * _BA
            pltpu.sync_copy(ei_hbm.at[pl.ds(E + base, _BA)], tails_v)
            pltpu.sync_copy(et_hbm.at[pl.ds(base, _BA)], gidx_v)

            def mix(j, _):
                t = tails_v[pl.ds(j * 16, 16)]
                ty = gidx_v[pl.ds(j * 16, 16)]
                gidx_v[pl.ds(j * 16, 16)] = ty * 4 + (t & 3)
                prow_v[pl.ds(j * 16, 16)] = lax.shift_right_logical(t, 2)
                return 0
            lax.fori_loop(0, _BA // 16, mix, 0)

            pltpu.async_copy(aug_hbm.at[gidx_v], rows_v, sem).wait()
            pltpu.sync_copy(rows_v, acc_sh.at[prow_v], add=True)
            return 0

        lax.fori_loop(0, EPW // _BA, chunk, 0)
        plsc.subcore_barrier()
        pltpu.sync_copy(acc_sh.at[pl.ds(sid * NQS, NQS)],
                        out_hbm.at[cid, pl.ds(sid * NQS, NQS)])

    return body(edge_ix_flat, edge_type, aug4)


_BC = 32
_NCH = E // _BC
_TRIPS = (_NCH + NW - 1) // NW + 1


def _edge_sc(edge_ix_flat, edge_type, p_tail, p_head, a_head, r_att, r_agg,
             av_flat):
    mesh = plsc.VectorSubcoreMesh(core_axis_name="c", subcore_axis_name="s")

    @functools.partial(
        pl.kernel,
        out_type=(jax.ShapeDtypeStruct((NC, N_PAD, DIN), jnp.float32),
                  jax.ShapeDtypeStruct((NC, NG, DIN), jnp.float32)),
        mesh=mesh,
        scratch_types=[
            pltpu.VMEM((2, _BC), jnp.int32),
            pltpu.VMEM((2, _BC), jnp.int32),
            pltpu.VMEM((2, _BC), jnp.int32),
            pltpu.VMEM((2, _BC), jnp.int32),
            pltpu.VMEM((2, _BC, DIN), jnp.float32),
            pltpu.VMEM((2, _BC, DIN), jnp.float32),
            pltpu.VMEM((2, _BC, DIN), jnp.float32),
            pltpu.VMEM((2, _BC, DIN), jnp.float32),
            pltpu.VMEM((DIN,), jnp.float32),
            pltpu.VMEM((_BC * NH,), jnp.float32),
            pltpu.VMEM((DIN,), jnp.float32),
            pltpu.VMEM_SHARED((N_PAD, DIN), jnp.float32),
            pltpu.VMEM_SHARED((NG, DIN), jnp.float32),
            pltpu.SemaphoreType.DMA,
            pltpu.SemaphoreType.DMA,
            pltpu.SemaphoreType.DMA,
            pltpu.SemaphoreType.DMA,
            pltpu.SemaphoreType.DMA,
            pltpu.SemaphoreType.DMA,
            pltpu.SemaphoreType.DMA,
            pltpu.SemaphoreType.DMA,
            pltpu.SemaphoreType.DMA,
            pltpu.SemaphoreType.DMA,
        ],
        compiler_params=pltpu.CompilerParams(needs_layout_passes=False),
    )
    def body(ei_hbm, et_hbm, ptail_hbm, phead_hbm, ahead_hbm, ratt_hbm,
             ragg_hbm, av_hbm, acc_out, s_out, tails_v, heads_v, types_v,
             tgrp_v, bufA, bufB, bufC, stage_s, logit_st, w_ch, av_v,
             acc_sh, s_sh, semi0, semi1, semA0, semA1, semB0, semB1,
             semC0, semC1, semS0, semS1):
        cid = lax.axis_index("c")
        sid = lax.axis_index("s")
        wid = sid * NC + cid
        zvec = jnp.zeros((16,), jnp.float32)
        iota = lax.iota(jnp.int32, 16)
        lane15 = iota == 15
        semi = (semi0, semi1)
        semA = (semA0, semA1)
        semB = (semB0, semB1)
        semC = (semC0, semC1)
        semS = (semS0, semS1)

        pltpu.sync_copy(av_hbm, av_v)

        def zstore(j, _):
            bufA[0, j // 8, pl.ds((j % 8) * 16, 16)] = zvec
            return 0
        lax.fori_loop(0, _BC * 8, zstore, 0)
        for q in range(NPS // _BC):
            pltpu.sync_copy(bufA.at[0],
                            acc_sh.at[pl.ds(sid * NPS + q * _BC, _BC)])
        pltpu.sync_copy(bufA.at[0], s_sh.at[pl.ds(sid * NGS, _BC)])
        pltpu.sync_copy(bufA.at[0, pl.ds(0, NGS - _BC)],
                        s_sh.at[pl.ds(sid * NGS + _BC, NGS - _BC)])
        plsc.subcore_barrier()

        def idx_issue(c, p):
            base = c * _BC
            pltpu.async_copy(ei_hbm.at[pl.ds(base, _BC)], heads_v.at[p],
                             semi[p])
            pltpu.async_copy(ei_hbm.at[pl.ds(E + base, _BC)], tails_v.at[p],
                             semi[p])
            pltpu.async_copy(et_hbm.at[pl.ds(base, _BC)], types_v.at[p],
                             semi[p])

        def idx_wait(p):
            for dst in (heads_v, tails_v, types_v):
                pltpu.make_async_copy(et_hbm.at[pl.ds(0, _BC)], dst.at[p],
                                      semi[p]).wait()

        def gather_issue(p):
            pltpu.async_copy(ptail_hbm.at[tails_v.at[p]], bufA.at[p], semA[p])
            pltpu.async_copy(phead_hbm.at[heads_v.at[p]], bufB.at[p], semB[p])
            pltpu.async_copy(ratt_hbm.at[types_v.at[p]], bufC.at[p], semC[p])

        def gather_wait(p):
            pltpu.make_async_copy(ptail_hbm.at[tails_v.at[p]], bufA.at[p],
                                  semA[p]).wait()
            pltpu.make_async_copy(phead_hbm.at[heads_v.at[p]], bufB.at[p],
                                  semB[p]).wait()
            pltpu.make_async_copy(ratt_hbm.at[types_v.at[p]], bufC.at[p],
                                  semC[p]).wait()

        def compute(p):
            for g in range(_BC // 16):
                @plsc.parallel_loop(0, 16, unroll=2)
                def p1(el):
                    e = g * 16 + el
                    for h in range(NH):
                        stage_s[p, e, pl.ds(h * 16, 16)] = zvec
                        u = (bufA[p, e, pl.ds(h * 16, 16)]
                             + bufB[p, e, pl.ds(h * 16, 16)]
                             + bufC[p, e, pl.ds(h * 16, 16)])
                        u = jnp.where(u >= 0, u, 0.2 * u)
                        t = u * av_v[pl.ds(h * 16, 16)]
                        cs = plsc.cumsum(t)
                        plsc.store_scatter(
                            logit_st,
                            [jnp.full((16,), h * 16 + el, jnp.int32)],
                            cs, mask=lane15)

                tl = tails_v[p, pl.ds(g * 16, 16)]
                tgrp_v[p, pl.ds(g * 16, 16)] = lax.shift_right_logical(tl, 4)
                scol = (tl & 15) * 8
                erow = g * 16 + iota
                for h in range(NH):
                    wv = jnp.exp(logit_st[pl.ds(h * 16, 16)])
                    w_ch[pl.ds((g * NH + h) * 16, 16)] = wv
                    plsc.store_scatter(stage_s.at[p], [erow, scol + h], wv)

            c4 = pltpu.async_copy(ahead_hbm.at[heads_v.at[p]], bufB.at[p],
                                  semB[p])
            c5 = pltpu.async_copy(ragg_hbm.at[types_v.at[p]], bufC.at[p],
                                  semC[p])
            c4.wait()
            c5.wait()

            @plsc.parallel_loop(0, _BC, unroll=2)
            def p3(e):
                g = lax.shift_right_logical(e, 4)
                el = e & 15
                for h in range(NH):
                    wsp = plsc.load_gather(
                        w_ch, [jnp.full((16,), (g * NH + h) * 16 + el,
                                        jnp.int32)])
                    v = (bufB[p, e, pl.ds(h * 16, 16)]
                         + bufC[p, e, pl.ds(h * 16, 16)])
                    bufA[p, e, pl.ds(h * 16, 16)] = v * wsp

            pltpu.async_copy(bufA.at[p], acc_sh.at[tails_v.at[p]], semS[p],
                             add=True)
            pltpu.async_copy(stage_s.at[p], s_sh.at[tgrp_v.at[p]], semS[p],
                             add=True)

        def scatter_wait(p):
            pltpu.make_async_copy(bufA.at[p], acc_sh.at[tails_v.at[p]],
                                  semS[p]).wait()
            pltpu.make_async_copy(stage_s.at[p], s_sh.at[tgrp_v.at[p]],
                                  semS[p]).wait()

        idx_issue(wid, 0)
        idx_wait(0)
        gather_issue(0)

        def outer(j2, _):
            for par in range(2):
                j = j2 * 2 + par
                c = wid + j * NW
                cn = wid + (j + 1) * NW

                @pl.when(cn < _NCH)
                def _():
                    @pl.when(j >= 1)
                    def _():
                        scatter_wait(par ^ 1)
                    idx_issue(cn, par ^ 1)

                @pl.when(c < _NCH)
                def _():
                    gather_wait(par)
                    compute(par)

                @pl.when(cn < _NCH)
                def _():
                    idx_wait(par ^ 1)
                    gather_issue(par ^ 1)
            return 0

        lax.fori_loop(0, (_TRIPS + 1) // 2, outer, 0)
        scatter_wait(0)
        scatter_wait(1)
        plsc.subcore_barrier()
        pltpu.sync_copy(acc_sh.at[pl.ds(sid * NPS, NPS)],
                        acc_out.at[cid, pl.ds(sid * NPS, NPS)])
        pltpu.sync_copy(s_sh.at[pl.ds(sid * NGS, NGS)],
                        s_out.at[cid, pl.ds(sid * NGS, NGS)])

    return body(edge_ix_flat, edge_type, p_tail, p_head, a_head, r_att,
                r_agg, av_flat)


def _dense_body(emb_ref, a0_ref, a1_ref, emb_rel_ref, WtT_ref, WhT_ref, WahT_ref,
                WrT_ref, WarT_ref, ba_ref, bg_ref, HSel_ref,
                ptail_ref, phead_ref, ahead_ref, lself_ref, vself_ref,
                ratt_ref, ragg_ref):
    e = emb_ref[...]
    f32 = jnp.float32
    pt = jnp.dot(e, WtT_ref[...], preferred_element_type=f32) + ba_ref[...]
    ph = jnp.dot(e, WhT_ref[...], preferred_element_type=f32)
    ah = jnp.dot(e, WahT_ref[...], preferred_element_type=f32) + bg_ref[...]
    asum = a0_ref[...] + a1_ref[...]
    sr = asum[:, :DREL] / asum[:, DREL:DREL + 1]
    satt = jnp.dot(sr, WrT_ref[...], preferred_element_type=f32)
    sagg = jnp.dot(sr, WarT_ref[...], preferred_element_type=f32)
    ptail_ref[...] = pt
    phead_ref[...] = ph
    ahead_ref[...] = ah
    us = _lrelu(pt + ph + satt)
    lself_ref[...] = jnp.dot(us, HSel_ref[...], preferred_element_type=f32)
    vself_ref[...] = ah + sagg
    er = emb_rel_ref[...]
    ratt_ref[...] = jnp.dot(er, WrT_ref[...], preferred_element_type=f32)
    ragg_ref[...] = jnp.dot(er, WarT_ref[...], preferred_element_type=f32)


def _dense_tables(emb_ent, a0, a1, emb_rel, W_attn, b_attn, attn_vec, W_aggr, b_aggr):
    f32 = jnp.float32
    WtT = W_attn[:, :DIN].T
    WhT = W_attn[:, DIN:2 * DIN].T
    WrT = W_attn[:, 2 * DIN:].T
    WahT = W_aggr[:, :DIN].T
    WarT = W_aggr[:, DIN:].T
    av = attn_vec.reshape(NH, DH)
    d_idx = jnp.arange(DIN)
    h_idx = jnp.arange(DH)
    HSel = jnp.where((d_idx[:, None] // DH) == h_idx[None, :],
                     av.reshape(-1)[d_idx][:, None], 0.0).astype(f32)
    grid = (N // ROWS_BLK,)
    full = lambda r, c: pl.BlockSpec((r, c), lambda i: (0, 0))
    blk = lambda c: pl.BlockSpec((ROWS_BLK, c), lambda i: (i, 0))
    return pl.pallas_call(
        _dense_body,
        grid=grid,
        in_specs=[blk(DIN), blk(2 * DREL), blk(2 * DREL), full(NREL, DREL),
                  full(DIN, DIN), full(DIN, DIN), full(DIN, DIN), full(DREL, DIN),
                  full(DREL, DIN), full(1, DIN), full(1, DIN), full(DIN, DH)],
        out_specs=[blk(DIN), blk(DIN), blk(DIN), blk(DH), blk(DIN),
                   full(NREL, DIN), full(NREL, DIN)],
        out_shape=[
            jax.ShapeDtypeStruct((N, DIN), f32),
            jax.ShapeDtypeStruct((N, DIN), f32),
            jax.ShapeDtypeStruct((N, DIN), f32),
            jax.ShapeDtypeStruct((N, DH), f32),
            jax.ShapeDtypeStruct((N, DIN), f32),
            jax.ShapeDtypeStruct((NREL, DIN), f32),
            jax.ShapeDtypeStruct((NREL, DIN), f32),
        ],
    )(emb_ent, a0, a1, emb_rel, WtT, WhT, WahT, WrT, WarT,
      b_attn.reshape(1, DIN), b_aggr.reshape(1, DIN), HSel)


def _combine_body(a0_ref, a1_ref, s_ref, lself_ref, vself_ref, rep_ref, out_ref):
    f32 = jnp.float32
    sw = jnp.dot(s_ref[...], rep_ref[...], preferred_element_type=f32)
    lw = jnp.dot(lself_ref[...], rep_ref[...], preferred_element_type=f32)
    z = jnp.exp(-lw)
    acc = a0_ref[...] + a1_ref[...]
    out_ref[...] = (acc * z + vself_ref[...]) / (1.0 + sw * z + 1e-16)


def _combine(acc0, acc1, s16, l_self, V_self):
    rep = (jnp.arange(DIN)[None, :] // DH == jnp.arange(DH)[:, None]).astype(jnp.float32)
    blk = lambda c: pl.BlockSpec((ROWS_BLK, c), lambda i: (i, 0))
    return pl.pallas_call(
        _combine_body,
        grid=(N // ROWS_BLK,),
        in_specs=[blk(DIN), blk(DIN), blk(DH), blk(DH), blk(DIN),
                  pl.BlockSpec((DH, DIN), lambda i: (0, 0))],
        out_specs=blk(DIN),
        out_shape=jax.ShapeDtypeStruct((N, DIN), jnp.float32),
    )(acc0, acc1, s16, l_self, V_self, rep)


def kernel(emb_ent, edge_index, edge_type, emb_rel, W_attn, b_attn, attn_vec, W_aggr, b_aggr):
    f32 = jnp.float32
    ei_flat = edge_index.reshape(-1)

    base_rel = jnp.concatenate(
        [emb_rel, jnp.ones((NREL, 1), f32), jnp.zeros((NREL, 15), f32)],
        axis=1)
    aug4 = jnp.einsum('qr,tc->tqrc', jnp.eye(4, dtype=f32),
                      base_rel).reshape(NREL * 4, DIN)
    acc_sr = _selfrel_sc(ei_flat, edge_type, aug4)
    acc_sr = acc_sr.reshape(NC, N_PAD, 2 * DREL)[:, :N]

    P_tail, P_head, A_head, l_self, V_self, R_att, R_agg = _dense_tables(
        emb_ent, acc_sr[0], acc_sr[1], emb_rel, W_attn, b_attn, attn_vec,
        W_aggr, b_aggr)

    acc_out, s_out = _edge_sc(ei_flat, edge_type, P_tail, P_head, A_head,
                              R_att, R_agg, attn_vec.reshape(-1))
    acc0 = acc_out[0, :N]
    acc1 = acc_out[1, :N]
    s8 = (s_out[0] + s_out[1]).reshape(N_PAD, NH)[:N]
    s16 = jnp.concatenate([s8, jnp.zeros((N, NH), f32)], axis=1)

    return _combine(acc0, acc1, s16, l_self, V_self)


# speedup vs baseline: 162.9449x; 1.0114x over previous
import functools
import jax
import jax.numpy as jnp
from jax import lax
from jax.experimental import pallas as pl
from jax.experimental.pallas import tpu as pltpu
from jax.experimental.pallas import tpu_sc as plsc

N = 10000
E = 320000
DIN = 128
DREL = 16
NREL = 256
NH = 8
DH = 16

NC = 2
NS = 16
NW = NC * NS
EPW = E // NW
N_PAD = 10240
NPS = N_PAD // NS
NQ = N_PAD // 4
NQS = NQ // NS
NG = N_PAD // 16
NGS = NG // NS

ROWS_BLK = 1000


def _lrelu(x):
    return jnp.where(x >= 0, x, 0.2 * x)


_BA = 400
_NCHA = EPW // _BA


def _selfrel_sc(edge_ix_flat, edge_type, aug4):
    mesh = plsc.VectorSubcoreMesh(core_axis_name="c", subcore_axis_name="s")

    @functools.partial(
        pl.kernel,
        out_type=jax.ShapeDtypeStruct((NC, NQ, DIN), jnp.float32),
        mesh=mesh,
        scratch_types=[
            pltpu.VMEM((_BA,), jnp.int32),
            pltpu.VMEM((_BA,), jnp.int32),
            pltpu.VMEM((_BA,), jnp.int32),
            pltpu.VMEM((_BA,), jnp.int32),
            pltpu.VMEM((_BA,), jnp.int32),
            pltpu.VMEM((_BA,), jnp.int32),
            pltpu.VMEM((2, _BA, DIN), jnp.float32),
            pltpu.VMEM_SHARED((NQ, DIN), jnp.float32),
            pltpu.SemaphoreType.DMA,
            pltpu.SemaphoreType.DMA,
            pltpu.SemaphoreType.DMA,
            pltpu.SemaphoreType.DMA,
        ],
        compiler_params=pltpu.CompilerParams(needs_layout_passes=False),
    )
    def body(ei_hbm, et_hbm, aug_hbm, out_hbm, tails0_v, tails1_v, gidx0_v,
             gidx1_v, prow0_v, prow1_v, rows_v, acc_sh, semG0, semG1,
             semS0, semS1):
        cid = lax.axis_index("c")
        sid = lax.axis_index("s")
        wid = sid * NC + cid
        zvec = jnp.zeros((16,), jnp.float32)
        semG = (semG0, semG1)
        semS = (semS0, semS1)
        tails_l = (tails0_v, tails1_v)
        gidx_l = (gidx0_v, gidx1_v)
        prow_l = (prow0_v, prow1_v)

        def zstore(j, _):
            rows_v[0, j // 8, pl.ds((j % 8) * 16, 16)] = zvec
            return 0
        lax.fori_loop(0, NQS * 8, zstore, 0)
        pltpu.sync_copy(rows_v.at[0, pl.ds(0, NQS)],
                        acc_sh.at[pl.ds(sid * NQS, NQS)])
        plsc.subcore_barrier()

        def idx_mix(k, p):
            base = wid * EPW + k * _BA
            pltpu.sync_copy(ei_hbm.at[pl.ds(E + base, _BA)], tails_l[p])
            pltpu.sync_copy(et_hbm.at[pl.ds(base, _BA)], gidx_l[p])

            def mix(j, _):
                t = tails_l[p][pl.ds(j * 16, 16)]
                ty = gidx_l[p][pl.ds(j * 16, 16)]
                gidx_l[p][pl.ds(j * 16, 16)] = ty * 4 + (t & 3)
                prow_l[p][pl.ds(j * 16, 16)] = lax.shift_right_logical(t, 2)
                return 0
            lax.fori_loop(0, _BA // 16, mix, 0)

        def gather_issue(p):
            pltpu.async_copy(aug_hbm.at[gidx_l[p]], rows_v.at[p], semG[p])

        def gather_wait(p):
            pltpu.make_async_copy(aug_hbm.at[gidx_l[p]], rows_v.at[p],
                                  semG[p]).wait()

        def scatter_wait(p):
            pltpu.make_async_copy(rows_v.at[p], acc_sh.at[prow_l[p]],
                                  semS[p]).wait()

        idx_mix(0, 0)
        gather_issue(0)

        def outer(kk, _):
            for p in range(2):
                k = kk * 2 + p

                @pl.when(k + 1 < _NCHA)
                def _():
                    @pl.when(k >= 1)
                    def _():
                        scatter_wait(p ^ 1)
                    idx_mix(k + 1, p ^ 1)

                @pl.when(k < _NCHA)
                def _():
                    gather_wait(p)
                    pltpu.async_copy(rows_v.at[p], acc_sh.at[prow_l[p]],
                                     semS[p], add=True)

                @pl.when(k + 1 < _NCHA)
                def _():
                    gather_issue(p ^ 1)
            return 0

        lax.fori_loop(0, (_NCHA + 1) // 2, outer, 0)
        scatter_wait(0)
        scatter_wait(1)
        plsc.subcore_barrier()
        pltpu.sync_copy(acc_sh.at[pl.ds(sid * NQS, NQS)],
                        out_hbm.at[cid, pl.ds(sid * NQS, NQS)])

    return body(edge_ix_flat, edge_type, aug4)


_BC = 32
_NCH = E // _BC
_TRIPS = (_NCH + NW - 1) // NW + 1


def _edge_sc(edge_ix_flat, edge_type, p_tail, p_head, a_head, r_att, r_agg,
             av_flat):
    mesh = plsc.VectorSubcoreMesh(core_axis_name="c", subcore_axis_name="s")

    @functools.partial(
        pl.kernel,
        out_type=(jax.ShapeDtypeStruct((NC, N_PAD, DIN), jnp.float32),
                  jax.ShapeDtypeStruct((NC, NG, DIN), jnp.float32)),
        mesh=mesh,
        scratch_types=[
            pltpu.VMEM((2, _BC), jnp.int32),
            pltpu.VMEM((2, _BC), jnp.int32),
            pltpu.VMEM((2, _BC), jnp.int32),
            pltpu.VMEM((2, _BC), jnp.int32),
            pltpu.VMEM((2, _BC, DIN), jnp.float32),
            pltpu.VMEM((2, _BC, DIN), jnp.float32),
            pltpu.VMEM((2, _BC, DIN), jnp.float32),
            pltpu.VMEM((2, _BC, DIN), jnp.float32),
            pltpu.VMEM((2, _BC, DIN), jnp.float32),
            pltpu.VMEM((DIN,), jnp.float32),
            pltpu.VMEM((_BC * NH,), jnp.float32),
            pltpu.VMEM((DIN,), jnp.float32),
            pltpu.VMEM_SHARED((N_PAD, DIN), jnp.float32),
            pltpu.VMEM_SHARED((NG, DIN), jnp.float32),
            pltpu.SemaphoreType.DMA,
            pltpu.SemaphoreType.DMA,
            pltpu.SemaphoreType.DMA,
            pltpu.SemaphoreType.DMA,
            pltpu.SemaphoreType.DMA,
            pltpu.SemaphoreType.DMA,
            pltpu.SemaphoreType.DMA,
            pltpu.SemaphoreType.DMA,
            pltpu.SemaphoreType.DMA,
            pltpu.SemaphoreType.DMA,
        ],
        compiler_params=pltpu.CompilerParams(needs_layout_passes=False),
    )
    def body(ei_hbm, et_hbm, ptail_hbm, phead_hbm, ahead_hbm, ratt_hbm,
             ragg_hbm, av_hbm, acc_out, s_out, tails_v, heads_v, types_v,
             tgrp_v, bufA, bufB, bufC, bufD, stage_s, logit_st, w_ch, av_v,
             acc_sh, s_sh, semi0, semi1, semA0, semA1, semB0, semB1,
             semC0, semC1, semS0, semS1):
        cid = lax.axis_index("c")
        sid = lax.axis_index("s")
        wid = sid * NC + cid
        zvec = jnp.zeros((16,), jnp.float32)
        iota = lax.iota(jnp.int32, 16)
        lane15 = iota == 15
        semi = (semi0, semi1)
        semA = (semA0, semA1)
        semB = (semB0, semB1)
        semC = (semC0, semC1)
        semS = (semS0, semS1)

        pltpu.sync_copy(av_hbm, av_v)

        def zstore(j, _):
            bufA[0, j // 8, pl.ds((j % 8) * 16, 16)] = zvec
            return 0
        lax.fori_loop(0, _BC * 8, zstore, 0)
        for q in range(NPS // _BC):
            pltpu.sync_copy(bufA.at[0],
                            acc_sh.at[pl.ds(sid * NPS + q * _BC, _BC)])
        pltpu.sync_copy(bufA.at[0], s_sh.at[pl.ds(sid * NGS, _BC)])
        pltpu.sync_copy(bufA.at[0, pl.ds(0, NGS - _BC)],
                        s_sh.at[pl.ds(sid * NGS + _BC, NGS - _BC)])
        plsc.subcore_barrier()

        def idx_issue(c, p):
            base = c * _BC
            pltpu.async_copy(ei_hbm.at[pl.ds(base, _BC)], heads_v.at[p],
                             semi[p])
            pltpu.async_copy(ei_hbm.at[pl.ds(E + base, _BC)], tails_v.at[p],
                             semi[p])
            pltpu.async_copy(et_hbm.at[pl.ds(base, _BC)], types_v.at[p],
                             semi[p])

        def idx_wait(p):
            for dst in (heads_v, tails_v, types_v):
                pltpu.make_async_copy(et_hbm.at[pl.ds(0, _BC)], dst.at[p],
                                      semi[p]).wait()

        def gather_issue(p):
            pltpu.async_copy(ptail_hbm.at[tails_v.at[p]], bufA.at[p], semA[p])
            pltpu.async_copy(ahead_hbm.at[heads_v.at[p]], bufD.at[p], semA[p])
            pltpu.async_copy(phead_hbm.at[heads_v.at[p]], bufB.at[p], semB[p])
            pltpu.async_copy(ratt_hbm.at[types_v.at[p]], bufC.at[p], semC[p])

        def gather_wait(p):
            pltpu.make_async_copy(ptail_hbm.at[tails_v.at[p]], bufA.at[p],
                                  semA[p]).wait()
            pltpu.make_async_copy(ahead_hbm.at[heads_v.at[p]], bufD.at[p],
                                  semA[p]).wait()
            pltpu.make_async_copy(phead_hbm.at[heads_v.at[p]], bufB.at[p],
                                  semB[p]).wait()
            pltpu.make_async_copy(ratt_hbm.at[types_v.at[p]], bufC.at[p],
                                  semC[p]).wait()

        def compute(p):
            for g in range(_BC // 16):
                @plsc.parallel_loop(0, 16, unroll=2)
                def p1(el):
                    e = g * 16 + el
                    for h in range(NH):
                        stage_s[p, e, pl.ds(h * 16, 16)] = zvec
                        u = (bufA[p, e, pl.ds(h * 16, 16)]
                             + bufB[p, e, pl.ds(h * 16, 16)]
                             + bufC[p, e, pl.ds(h * 16, 16)])
                        u = jnp.where(u >= 0, u, 0.2 * u)
                        t = u * av_v[pl.ds(h * 16, 16)]
                        cs = plsc.cumsum(t)
                        plsc.store_scatter(
                            logit_st,
                            [jnp.full((16,), h * 16 + el, jnp.int32)],
                            cs, mask=lane15)

                tl = tails_v[p, pl.ds(g * 16, 16)]
                tgrp_v[p, pl.ds(g * 16, 16)] = lax.shift_right_logical(tl, 4)
                scol = (tl & 15) * 8
                erow = g * 16 + iota
                for h in range(NH):
                    wv = jnp.exp(logit_st[pl.ds(h * 16, 16)])
                    w_ch[pl.ds((g * NH + h) * 16, 16)] = wv
                    plsc.store_scatter(stage_s.at[p], [erow, scol + h], wv)

            c5 = pltpu.async_copy(ragg_hbm.at[types_v.at[p]], bufC.at[p],
                                  semC[p])
            c5.wait()

            @plsc.parallel_loop(0, _BC, unroll=2)
            def p3(e):
                g = lax.shift_right_logical(e, 4)
                el = e & 15
                for h in range(NH):
                    wsp = plsc.load_gather(
                        w_ch, [jnp.full((16,), (g * NH + h) * 16 + el,
                                        jnp.int32)])
                    v = (bufD[p, e, pl.ds(h * 16, 16)]
                         + bufC[p, e, pl.ds(h * 16, 16)])
                    bufA[p, e, pl.ds(h * 16, 16)] = v * wsp

            pltpu.async_copy(bufA.at[p], acc_sh.at[tails_v.at[p]], semS[p],
                             add=True)
            pltpu.async_copy(stage_s.at[p], s_sh.at[tgrp_v.at[p]], semS[p],
                             add=True)

        def scatter_wait(p):
            pltpu.make_async_copy(bufA.at[p], acc_sh.at[tails_v.at[p]],
                                  semS[p]).wait()
            pltpu.make_async_copy(stage_s.at[p], s_sh.at[tgrp_v.at[p]],
                                  semS[p]).wait()

        idx_issue(wid, 0)
        idx_wait(0)
        gather_issue(0)

        def outer(j2, _):
            for par in range(2):
                j = j2 * 2 + par
                c = wid + j * NW
                cn = wid + (j + 1) * NW

                @pl.when(cn < _NCH)
                def _():
                    @pl.when(j >= 1)
                    def _():
                        scatter_wait(par ^ 1)
                    idx_issue(cn, par ^ 1)

                @pl.when(c < _NCH)
                def _():
                    gather_wait(par)
                    compute(par)

                @pl.when(cn < _NCH)
                def _():
                    idx_wait(par ^ 1)
                    gather_issue(par ^ 1)
            return 0

        lax.fori_loop(0, (_TRIPS + 1) // 2, outer, 0)
        scatter_wait(0)
        scatter_wait(1)
        plsc.subcore_barrier()
        pltpu.sync_copy(acc_sh.at[pl.ds(sid * NPS, NPS)],
                        acc_out.at[cid, pl.ds(sid * NPS, NPS)])
        pltpu.sync_copy(s_sh.at[pl.ds(sid * NGS, NGS)],
                        s_out.at[cid, pl.ds(sid * NGS, NGS)])

    return body(edge_ix_flat, edge_type, p_tail, p_head, a_head, r_att,
                r_agg, av_flat)


def _dense_body(emb_ref, a0_ref, a1_ref, emb_rel_ref, WtT_ref, WhT_ref, WahT_ref,
                WrT_ref, WarT_ref, ba_ref, bg_ref, HSel_ref,
                ptail_ref, phead_ref, ahead_ref, lself_ref, vself_ref,
                ratt_ref, ragg_ref):
    e = emb_ref[...]
    f32 = jnp.float32
    pt = jnp.dot(e, WtT_ref[...], preferred_element_type=f32) + ba_ref[...]
    ph = jnp.dot(e, WhT_ref[...], preferred_element_type=f32)
    ah = jnp.dot(e, WahT_ref[...], preferred_element_type=f32) + bg_ref[...]
    asum = a0_ref[...] + a1_ref[...]
    sr = asum[:, :DREL] / asum[:, DREL:DREL + 1]
    satt = jnp.dot(sr, WrT_ref[...], preferred_element_type=f32)
    sagg = jnp.dot(sr, WarT_ref[...], preferred_element_type=f32)
    ptail_ref[...] = pt
    phead_ref[...] = ph
    ahead_ref[...] = ah
    us = _lrelu(pt + ph + satt)
    lself_ref[...] = jnp.dot(us, HSel_ref[...], preferred_element_type=f32)
    vself_ref[...] = ah + sagg
    er = emb_rel_ref[...]
    ratt_ref[...] = jnp.dot(er, WrT_ref[...], preferred_element_type=f32)
    ragg_ref[...] = jnp.dot(er, WarT_ref[...], preferred_element_type=f32)


def _dense_tables(emb_ent, a0, a1, emb_rel, W_attn, b_attn, attn_vec, W_aggr, b_aggr):
    f32 = jnp.float32
    WtT = W_attn[:, :DIN].T
    WhT = W_attn[:, DIN:2 * DIN].T
    WrT = W_attn[:, 2 * DIN:].T
    WahT = W_aggr[:, :DIN].T
    WarT = W_aggr[:, DIN:].T
    av = attn_vec.reshape(NH, DH)
    d_idx = jnp.arange(DIN)
    h_idx = jnp.arange(DH)
    HSel = jnp.where((d_idx[:, None] // DH) == h_idx[None, :],
                     av.reshape(-1)[d_idx][:, None], 0.0).astype(f32)
    grid = (N // ROWS_BLK,)
    full = lambda r, c: pl.BlockSpec((r, c), lambda i: (0, 0))
    blk = lambda c: pl.BlockSpec((ROWS_BLK, c), lambda i: (i, 0))
    return pl.pallas_call(
        _dense_body,
        grid=grid,
        in_specs=[blk(DIN), blk(2 * DREL), blk(2 * DREL), full(NREL, DREL),
                  full(DIN, DIN), full(DIN, DIN), full(DIN, DIN), full(DREL, DIN),
                  full(DREL, DIN), full(1, DIN), full(1, DIN), full(DIN, DH)],
        out_specs=[blk(DIN), blk(DIN), blk(DIN), blk(DH), blk(DIN),
                   full(NREL, DIN), full(NREL, DIN)],
        out_shape=[
            jax.ShapeDtypeStruct((N, DIN), f32),
            jax.ShapeDtypeStruct((N, DIN), f32),
            jax.ShapeDtypeStruct((N, DIN), f32),
            jax.ShapeDtypeStruct((N, DH), f32),
            jax.ShapeDtypeStruct((N, DIN), f32),
            jax.ShapeDtypeStruct((NREL, DIN), f32),
            jax.ShapeDtypeStruct((NREL, DIN), f32),
        ],
    )(emb_ent, a0, a1, emb_rel, WtT, WhT, WahT, WrT, WarT,
      b_attn.reshape(1, DIN), b_aggr.reshape(1, DIN), HSel)


def _combine_body(a0_ref, a1_ref, s_ref, lself_ref, vself_ref, rep_ref, out_ref):
    f32 = jnp.float32
    sw = jnp.dot(s_ref[...], rep_ref[...], preferred_element_type=f32)
    lw = jnp.dot(lself_ref[...], rep_ref[...], preferred_element_type=f32)
    z = jnp.exp(-lw)
    acc = a0_ref[...] + a1_ref[...]
    out_ref[...] = (acc * z + vself_ref[...]) / (1.0 + sw * z + 1e-16)


def _combine(acc0, acc1, s16, l_self, V_self):
    rep = (jnp.arange(DIN)[None, :] // DH == jnp.arange(DH)[:, None]).astype(jnp.float32)
    blk = lambda c: pl.BlockSpec((ROWS_BLK, c), lambda i: (i, 0))
    return pl.pallas_call(
        _combine_body,
        grid=(N // ROWS_BLK,),
        in_specs=[blk(DIN), blk(DIN), blk(DH), blk(DH), blk(DIN),
                  pl.BlockSpec((DH, DIN), lambda i: (0, 0))],
        out_specs=blk(DIN),
        out_shape=jax.ShapeDtypeStruct((N, DIN), jnp.float32),
    )(acc0, acc1, s16, l_self, V_self, rep)


def kernel(emb_ent, edge_index, edge_type, emb_rel, W_attn, b_attn, attn_vec, W_aggr, b_aggr):
    f32 = jnp.float32
    ei_flat = edge_index.reshape(-1)

    base_rel = jnp.concatenate(
        [emb_rel, jnp.ones((NREL, 1), f32), jnp.zeros((NREL, 15), f32)],
        axis=1)
    aug4 = jnp.einsum('qr,tc->tqrc', jnp.eye(4, dtype=f32),
                      base_rel).reshape(NREL * 4, DIN)
    acc_sr = _selfrel_sc(ei_flat, edge_type, aug4)
    acc_sr = acc_sr.reshape(NC, N_PAD, 2 * DREL)[:, :N]

    P_tail, P_head, A_head, l_self, V_self, R_att, R_agg = _dense_tables(
        emb_ent, acc_sr[0], acc_sr[1], emb_rel, W_attn, b_attn, attn_vec,
        W_aggr, b_aggr)

    acc_out, s_out = _edge_sc(ei_flat, edge_type, P_tail, P_head, A_head,
                              R_att, R_agg, attn_vec.reshape(-1))
    acc0 = acc_out[0, :N]
    acc1 = acc_out[1, :N]
    s8 = (s_out[0] + s_out[1]).reshape(N_PAD, NH)[:N]
    s16 = jnp.concatenate([s8, jnp.zeros((N, NH), f32)], axis=1)

    return _combine(acc0, acc1, s16, l_self, V_self)


# speedup vs baseline: 164.6897x; 1.0107x over previous
import functools
import jax
import jax.numpy as jnp
from jax import lax
from jax.experimental import pallas as pl
from jax.experimental.pallas import tpu as pltpu
from jax.experimental.pallas import tpu_sc as plsc

N = 10000
E = 320000
DIN = 128
DREL = 16
NREL = 256
NH = 8
DH = 16

NC = 2
NS = 16
NW = NC * NS
EPW = E // NW
N_PAD = 10240
NPS = N_PAD // NS
NQ = N_PAD // 4
NQS = NQ // NS
NG = N_PAD // 16
NGS = NG // NS

ROWS_BLK = 1000


def _lrelu(x):
    return jnp.where(x >= 0, x, 0.2 * x)


_BA = 400
_NCHA = EPW // _BA


def _selfrel_sc(edge_ix_flat, edge_type, aug4):
    mesh = plsc.VectorSubcoreMesh(core_axis_name="c", subcore_axis_name="s")

    @functools.partial(
        pl.kernel,
        out_type=jax.ShapeDtypeStruct((NC, NQ, DIN), jnp.float32),
        mesh=mesh,
        scratch_types=[
            pltpu.VMEM((_BA,), jnp.int32),
            pltpu.VMEM((_BA,), jnp.int32),
            pltpu.VMEM((_BA,), jnp.int32),
            pltpu.VMEM((_BA,), jnp.int32),
            pltpu.VMEM((_BA,), jnp.int32),
            pltpu.VMEM((_BA,), jnp.int32),
            pltpu.VMEM((2, _BA, DIN), jnp.float32),
            pltpu.VMEM_SHARED((NQ, DIN), jnp.float32),
            pltpu.SemaphoreType.DMA,
            pltpu.SemaphoreType.DMA,
            pltpu.SemaphoreType.DMA,
            pltpu.SemaphoreType.DMA,
        ],
        compiler_params=pltpu.CompilerParams(needs_layout_passes=False),
    )
    def body(ei_hbm, et_hbm, aug_hbm, out_hbm, tails0_v, tails1_v, gidx0_v,
             gidx1_v, prow0_v, prow1_v, rows_v, acc_sh, semG0, semG1,
             semS0, semS1):
        cid = lax.axis_index("c")
        sid = lax.axis_index("s")
        wid = sid * NC + cid
        zvec = jnp.zeros((16,), jnp.float32)
        semG = (semG0, semG1)
        semS = (semS0, semS1)
        tails_l = (tails0_v, tails1_v)
        gidx_l = (gidx0_v, gidx1_v)
        prow_l = (prow0_v, prow1_v)

        def zstore(j, _):
            rows_v[0, j // 8, pl.ds((j % 8) * 16, 16)] = zvec
            return 0
        lax.fori_loop(0, NQS * 8, zstore, 0)
        pltpu.sync_copy(rows_v.at[0, pl.ds(0, NQS)],
                        acc_sh.at[pl.ds(sid * NQS, NQS)])
        plsc.subcore_barrier()

        def idx_mix(k, p):
            base = wid * EPW + k * _BA
            pltpu.sync_copy(ei_hbm.at[pl.ds(E + base, _BA)], tails_l[p])
            pltpu.sync_copy(et_hbm.at[pl.ds(base, _BA)], gidx_l[p])

            def mix(j, _):
                t = tails_l[p][pl.ds(j * 16, 16)]
                ty = gidx_l[p][pl.ds(j * 16, 16)]
                gidx_l[p][pl.ds(j * 16, 16)] = ty * 4 + (t & 3)
                prow_l[p][pl.ds(j * 16, 16)] = lax.shift_right_logical(t, 2)
                return 0
            lax.fori_loop(0, _BA // 16, mix, 0)

        def gather_issue(p):
            pltpu.async_copy(aug_hbm.at[gidx_l[p]], rows_v.at[p], semG[p])

        def gather_wait(p):
            pltpu.make_async_copy(aug_hbm.at[gidx_l[p]], rows_v.at[p],
                                  semG[p]).wait()

        def scatter_wait(p):
            pltpu.make_async_copy(rows_v.at[p], acc_sh.at[prow_l[p]],
                                  semS[p]).wait()

        idx_mix(0, 0)
        gather_issue(0)

        def outer(kk, _):
            for p in range(2):
                k = kk * 2 + p

                @pl.when(k + 1 < _NCHA)
                def _():
                    @pl.when(k >= 1)
                    def _():
                        scatter_wait(p ^ 1)
                    idx_mix(k + 1, p ^ 1)

                @pl.when(k < _NCHA)
                def _():
                    gather_wait(p)
                    pltpu.async_copy(rows_v.at[p], acc_sh.at[prow_l[p]],
                                     semS[p], add=True)

                @pl.when(k + 1 < _NCHA)
                def _():
                    gather_issue(p ^ 1)
            return 0

        lax.fori_loop(0, (_NCHA + 1) // 2, outer, 0)
        scatter_wait(0)
        scatter_wait(1)
        plsc.subcore_barrier()
        pltpu.sync_copy(acc_sh.at[pl.ds(sid * NQS, NQS)],
                        out_hbm.at[cid, pl.ds(sid * NQS, NQS)])

    return body(edge_ix_flat, edge_type, aug4)


_BC = 32
_NCH = E // _BC
_TRIPS = (_NCH + NW - 1) // NW + 1


def _edge_sc(edge_ix_flat, edge_type, p_tail, p_head, a_head, r_att, r_agg,
             av_flat):
    mesh = plsc.VectorSubcoreMesh(core_axis_name="c", subcore_axis_name="s")

    @functools.partial(
        pl.kernel,
        out_type=(jax.ShapeDtypeStruct((NC, N_PAD, DIN), jnp.float32),
                  jax.ShapeDtypeStruct((NC, NG, DIN), jnp.float32)),
        mesh=mesh,
        scratch_types=[
            pltpu.VMEM((2, _BC), jnp.int32),
            pltpu.VMEM((2, _BC), jnp.int32),
            pltpu.VMEM((2, _BC), jnp.int32),
            pltpu.VMEM((2, _BC), jnp.int32),
            pltpu.VMEM((2, _BC, DIN), jnp.float32),
            pltpu.VMEM((2, _BC, DIN), jnp.float32),
            pltpu.VMEM((2, _BC, DIN), jnp.float32),
            pltpu.VMEM((2, _BC, DIN), jnp.float32),
            pltpu.VMEM((2, _BC, DIN), jnp.float32),
            pltpu.VMEM((DIN,), jnp.float32),
            pltpu.VMEM((_BC * NH,), jnp.float32),
            pltpu.VMEM((DIN,), jnp.float32),
            pltpu.VMEM_SHARED((N_PAD, DIN), jnp.float32),
            pltpu.VMEM_SHARED((NG, DIN), jnp.float32),
            pltpu.SemaphoreType.DMA,
            pltpu.SemaphoreType.DMA,
            pltpu.SemaphoreType.DMA,
            pltpu.SemaphoreType.DMA,
            pltpu.SemaphoreType.DMA,
            pltpu.SemaphoreType.DMA,
            pltpu.SemaphoreType.DMA,
            pltpu.SemaphoreType.DMA,
            pltpu.SemaphoreType.DMA,
            pltpu.SemaphoreType.DMA,
        ],
        compiler_params=pltpu.CompilerParams(needs_layout_passes=False),
    )
    def body(ei_hbm, et_hbm, ptail_hbm, phead_hbm, ahead_hbm, ratt_hbm,
             ragg_hbm, av_hbm, acc_out, s_out, tails_v, heads_v, types_v,
             tgrp_v, bufA, bufB, bufC, bufD, stage_s, logit_st, w_ch, av_v,
             acc_sh, s_sh, semi0, semi1, semA0, semA1, semB0, semB1,
             semC0, semC1, semS0, semS1):
        cid = lax.axis_index("c")
        sid = lax.axis_index("s")
        wid = sid * NC + cid
        zvec = jnp.zeros((16,), jnp.float32)
        iota = lax.iota(jnp.int32, 16)
        lane15 = iota == 15
        semi = (semi0, semi1)
        semA = (semA0, semA1)
        semB = (semB0, semB1)
        semC = (semC0, semC1)
        semS = (semS0, semS1)

        pltpu.sync_copy(av_hbm, av_v)

        def zstore(j, _):
            bufA[0, j // 8, pl.ds((j % 8) * 16, 16)] = zvec
            return 0
        lax.fori_loop(0, _BC * 8, zstore, 0)
        for q in range(NPS // _BC):
            pltpu.sync_copy(bufA.at[0],
                            acc_sh.at[pl.ds(sid * NPS + q * _BC, _BC)])
        pltpu.sync_copy(bufA.at[0], s_sh.at[pl.ds(sid * NGS, _BC)])
        pltpu.sync_copy(bufA.at[0, pl.ds(0, NGS - _BC)],
                        s_sh.at[pl.ds(sid * NGS + _BC, NGS - _BC)])
        plsc.subcore_barrier()

        def idx_issue(c, p):
            base = c * _BC
            pltpu.async_copy(ei_hbm.at[pl.ds(base, _BC)], heads_v.at[p],
                             semi[p])
            pltpu.async_copy(ei_hbm.at[pl.ds(E + base, _BC)], tails_v.at[p],
                             semi[p])
            pltpu.async_copy(et_hbm.at[pl.ds(base, _BC)], types_v.at[p],
                             semi[p])

        def idx_wait(p):
            for dst in (heads_v, tails_v, types_v):
                pltpu.make_async_copy(et_hbm.at[pl.ds(0, _BC)], dst.at[p],
                                      semi[p]).wait()

        def gather_issue(p):
            pltpu.async_copy(ptail_hbm.at[tails_v.at[p]], bufA.at[p], semA[p])
            pltpu.async_copy(ahead_hbm.at[heads_v.at[p]], bufD.at[p], semA[p])
            pltpu.async_copy(phead_hbm.at[heads_v.at[p]], bufB.at[p], semB[p])
            pltpu.async_copy(ratt_hbm.at[types_v.at[p]], bufC.at[p], semC[p])

        def gather_wait(p):
            pltpu.make_async_copy(ptail_hbm.at[tails_v.at[p]], bufA.at[p],
                                  semA[p]).wait()
            pltpu.make_async_copy(ahead_hbm.at[heads_v.at[p]], bufD.at[p],
                                  semA[p]).wait()
            pltpu.make_async_copy(phead_hbm.at[heads_v.at[p]], bufB.at[p],
                                  semB[p]).wait()
            pltpu.make_async_copy(ratt_hbm.at[types_v.at[p]], bufC.at[p],
                                  semC[p]).wait()

        def compute(p):
            for g in range(_BC // 16):
                @plsc.parallel_loop(0, 16, unroll=4)
                def p1(el):
                    e = g * 16 + el
                    for h in range(NH):
                        stage_s[p, e, pl.ds(h * 16, 16)] = zvec
                        u = (bufA[p, e, pl.ds(h * 16, 16)]
                             + bufB[p, e, pl.ds(h * 16, 16)]
                             + bufC[p, e, pl.ds(h * 16, 16)])
                        u = jnp.where(u >= 0, u, 0.2 * u)
                        t = u * av_v[pl.ds(h * 16, 16)]
                        cs = plsc.cumsum(t)
                        plsc.store_scatter(
                            logit_st,
                            [jnp.full((16,), h * 16 + el, jnp.int32)],
                            cs, mask=lane15)

                tl = tails_v[p, pl.ds(g * 16, 16)]
                tgrp_v[p, pl.ds(g * 16, 16)] = lax.shift_right_logical(tl, 4)
                scol = (tl & 15) * 8
                erow = g * 16 + iota
                for h in range(NH):
                    wv = jnp.exp(logit_st[pl.ds(h * 16, 16)])
                    w_ch[pl.ds((g * NH + h) * 16, 16)] = wv
                    plsc.store_scatter(stage_s.at[p], [erow, scol + h], wv)

            c5 = pltpu.async_copy(ragg_hbm.at[types_v.at[p]], bufC.at[p],
                                  semC[p])
            c5.wait()

            @plsc.parallel_loop(0, _BC, unroll=4)
            def p3(e):
                g = lax.shift_right_logical(e, 4)
                el = e & 15
                for h in range(NH):
                    wsp = plsc.load_gather(
                        w_ch, [jnp.full((16,), (g * NH + h) * 16 + el,
                                        jnp.int32)])
                    v = (bufD[p, e, pl.ds(h * 16, 16)]
                         + bufC[p, e, pl.ds(h * 16, 16)])
                    bufA[p, e, pl.ds(h * 16, 16)] = v * wsp

            pltpu.async_copy(bufA.at[p], acc_sh.at[tails_v.at[p]], semS[p],
                             add=True)
            pltpu.async_copy(stage_s.at[p], s_sh.at[tgrp_v.at[p]], semS[p],
                             add=True)

        def scatter_wait(p):
            pltpu.make_async_copy(bufA.at[p], acc_sh.at[tails_v.at[p]],
                                  semS[p]).wait()
            pltpu.make_async_copy(stage_s.at[p], s_sh.at[tgrp_v.at[p]],
                                  semS[p]).wait()

        idx_issue(wid, 0)
        idx_wait(0)
        gather_issue(0)

        def outer(j2, _):
            for par in range(2):
                j = j2 * 2 + par
                c = wid + j * NW
                cn = wid + (j + 1) * NW

                @pl.when(cn < _NCH)
                def _():
                    @pl.when(j >= 1)
                    def _():
                        scatter_wait(par ^ 1)
                    idx_issue(cn, par ^ 1)

                @pl.when(c < _NCH)
                def _():
                    gather_wait(par)
                    compute(par)

                @pl.when(cn < _NCH)
                def _():
                    idx_wait(par ^ 1)
                    gather_issue(par ^ 1)
            return 0

        lax.fori_loop(0, (_TRIPS + 1) // 2, outer, 0)
        scatter_wait(0)
        scatter_wait(1)
        plsc.subcore_barrier()
        pltpu.sync_copy(acc_sh.at[pl.ds(sid * NPS, NPS)],
                        acc_out.at[cid, pl.ds(sid * NPS, NPS)])
        pltpu.sync_copy(s_sh.at[pl.ds(sid * NGS, NGS)],
                        s_out.at[cid, pl.ds(sid * NGS, NGS)])

    return body(edge_ix_flat, edge_type, p_tail, p_head, a_head, r_att,
                r_agg, av_flat)


def _dense_body(emb_ref, a0_ref, a1_ref, emb_rel_ref, WtT_ref, WhT_ref, WahT_ref,
                WrT_ref, WarT_ref, ba_ref, bg_ref, HSel_ref,
                ptail_ref, phead_ref, ahead_ref, lself_ref, vself_ref,
                ratt_ref, ragg_ref):
    e = emb_ref[...]
    f32 = jnp.float32
    pt = jnp.dot(e, WtT_ref[...], preferred_element_type=f32) + ba_ref[...]
    ph = jnp.dot(e, WhT_ref[...], preferred_element_type=f32)
    ah = jnp.dot(e, WahT_ref[...], preferred_element_type=f32) + bg_ref[...]
    asum = a0_ref[...] + a1_ref[...]
    sr = asum[:, :DREL] / asum[:, DREL:DREL + 1]
    satt = jnp.dot(sr, WrT_ref[...], preferred_element_type=f32)
    sagg = jnp.dot(sr, WarT_ref[...], preferred_element_type=f32)
    ptail_ref[...] = pt
    phead_ref[...] = ph
    ahead_ref[...] = ah
    us = _lrelu(pt + ph + satt)
    lself_ref[...] = jnp.dot(us, HSel_ref[...], preferred_element_type=f32)
    vself_ref[...] = ah + sagg
    er = emb_rel_ref[...]
    ratt_ref[...] = jnp.dot(er, WrT_ref[...], preferred_element_type=f32)
    ragg_ref[...] = jnp.dot(er, WarT_ref[...], preferred_element_type=f32)


def _dense_tables(emb_ent, a0, a1, emb_rel, W_attn, b_attn, attn_vec, W_aggr, b_aggr):
    f32 = jnp.float32
    WtT = W_attn[:, :DIN].T
    WhT = W_attn[:, DIN:2 * DIN].T
    WrT = W_attn[:, 2 * DIN:].T
    WahT = W_aggr[:, :DIN].T
    WarT = W_aggr[:, DIN:].T
    av = attn_vec.reshape(NH, DH)
    d_idx = jnp.arange(DIN)
    h_idx = jnp.arange(DH)
    HSel = jnp.where((d_idx[:, None] // DH) == h_idx[None, :],
                     av.reshape(-1)[d_idx][:, None], 0.0).astype(f32)
    grid = (N // ROWS_BLK,)
    full = lambda r, c: pl.BlockSpec((r, c), lambda i: (0, 0))
    blk = lambda c: pl.BlockSpec((ROWS_BLK, c), lambda i: (i, 0))
    return pl.pallas_call(
        _dense_body,
        grid=grid,
        in_specs=[blk(DIN), blk(2 * DREL), blk(2 * DREL), full(NREL, DREL),
                  full(DIN, DIN), full(DIN, DIN), full(DIN, DIN), full(DREL, DIN),
                  full(DREL, DIN), full(1, DIN), full(1, DIN), full(DIN, DH)],
        out_specs=[blk(DIN), blk(DIN), blk(DIN), blk(DH), blk(DIN),
                   full(NREL, DIN), full(NREL, DIN)],
        out_shape=[
            jax.ShapeDtypeStruct((N, DIN), f32),
            jax.ShapeDtypeStruct((N, DIN), f32),
            jax.ShapeDtypeStruct((N, DIN), f32),
            jax.ShapeDtypeStruct((N, DH), f32),
            jax.ShapeDtypeStruct((N, DIN), f32),
            jax.ShapeDtypeStruct((NREL, DIN), f32),
            jax.ShapeDtypeStruct((NREL, DIN), f32),
        ],
    )(emb_ent, a0, a1, emb_rel, WtT, WhT, WahT, WrT, WarT,
      b_attn.reshape(1, DIN), b_aggr.reshape(1, DIN), HSel)


def _combine_body(a0_ref, a1_ref, s_ref, lself_ref, vself_ref, rep_ref, out_ref):
    f32 = jnp.float32
    sw = jnp.dot(s_ref[...], rep_ref[...], preferred_element_type=f32)
    lw = jnp.dot(lself_ref[...], rep_ref[...], preferred_element_type=f32)
    z = jnp.exp(-lw)
    acc = a0_ref[...] + a1_ref[...]
    out_ref[...] = (acc * z + vself_ref[...]) / (1.0 + sw * z + 1e-16)


def _combine(acc0, acc1, s16, l_self, V_self):
    rep = (jnp.arange(DIN)[None, :] // DH == jnp.arange(DH)[:, None]).astype(jnp.float32)
    blk = lambda c: pl.BlockSpec((ROWS_BLK, c), lambda i: (i, 0))
    return pl.pallas_call(
        _combine_body,
        grid=(N // ROWS_BLK,),
        in_specs=[blk(DIN), blk(DIN), blk(DH), blk(DH), blk(DIN),
                  pl.BlockSpec((DH, DIN), lambda i: (0, 0))],
        out_specs=blk(DIN),
        out_shape=jax.ShapeDtypeStruct((N, DIN), jnp.float32),
    )(acc0, acc1, s16, l_self, V_self, rep)


def kernel(emb_ent, edge_index, edge_type, emb_rel, W_attn, b_attn, attn_vec, W_aggr, b_aggr):
    f32 = jnp.float32
    ei_flat = edge_index.reshape(-1)

    base_rel = jnp.concatenate(
        [emb_rel, jnp.ones((NREL, 1), f32), jnp.zeros((NREL, 15), f32)],
        axis=1)
    aug4 = jnp.einsum('qr,tc->tqrc', jnp.eye(4, dtype=f32),
                      base_rel).reshape(NREL * 4, DIN)
    acc_sr = _selfrel_sc(ei_flat, edge_type, aug4)
    acc_sr = acc_sr.reshape(NC, N_PAD, 2 * DREL)[:, :N]

    P_tail, P_head, A_head, l_self, V_self, R_att, R_agg = _dense_tables(
        emb_ent, acc_sr[0], acc_sr[1], emb_rel, W_attn, b_attn, attn_vec,
        W_aggr, b_aggr)

    acc_out, s_out = _edge_sc(ei_flat, edge_type, P_tail, P_head, A_head,
                              R_att, R_agg, attn_vec.reshape(-1))
    acc0 = acc_out[0, :N]
    acc1 = acc_out[1, :N]
    s8 = (s_out[0] + s_out[1]).reshape(N_PAD, NH)[:N]
    s16 = jnp.concatenate([s8, jnp.zeros((N, NH), f32)], axis=1)

    return _combine(acc0, acc1, s16, l_self, V_self)
